# baseline (speedup 1.0000x reference)
import functools

import jax
import jax.numpy as jnp
from jax import lax
from jax.experimental import pallas as pl
from jax.experimental.pallas import tpu as pltpu
from jax.experimental.pallas import tpu_sc as plsc

NC = 2
NS = 16
NW = NC * NS
L = 16

COLS = 1024
CROWS = 8
B1 = 4096
B2 = 4096
B3 = 256
UNROLL = 16


def _start_pair(xh, yh, xb, yb, sx, sy, b, row0):
    pltpu.async_copy(xh.at[pl.ds(row0, CROWS)], xb.at[b], sx)
    pltpu.async_copy(yh.at[pl.ds(row0, CROWS)], yb.at[b], sy)


def _wait_pair(xh, yh, xb, yb, sx, sy, b):
    pltpu.make_async_copy(xh.at[pl.ds(0, CROWS)], xb.at[b], sx).wait()
    pltpu.make_async_copy(yh.at[pl.ds(0, CROWS)], yb.at[b], sy).wait()


def _zero_i32(ref, n):
    z = jnp.zeros((L,), jnp.int32)

    @pl.loop(0, n // L, unroll=16)
    def _(i):
        ref[pl.ds(i * L, L)] = z


def _zero_f32(ref, n):
    z = jnp.zeros((L,), jnp.float32)

    @pl.loop(0, n // L, unroll=16)
    def _(i):
        ref[pl.ds(i * L, L)] = z


def _mesh():
    return plsc.VectorSubcoreMesh(
        core_axis_name="c", subcore_axis_name="s", num_cores=NC, num_subcores=NS
    )


def _sc_pass1(nrows):
    rows_per_tile = nrows // NW
    nchunk = rows_per_tile // CROWS
    csteps = COLS // L

    @functools.partial(
        pl.kernel,
        out_type=jax.ShapeDtypeStruct((NW, L * B1), jnp.int32),
        mesh=_mesh(),
        compiler_params=pltpu.CompilerParams(needs_layout_passes=False),
        scratch_types=[
            pltpu.VMEM((2, CROWS, COLS), jnp.float32),
            pltpu.VMEM((2, CROWS, COLS), jnp.float32),
            pltpu.VMEM((L * B1,), jnp.int32),
            pltpu.SemaphoreType.DMA,
            pltpu.SemaphoreType.DMA,
            pltpu.SemaphoreType.DMA,
            pltpu.SemaphoreType.DMA,
        ],
    )
    def body(xh, yh, out_h, xb, yb, hist, sx0, sx1, sy0, sy1):
        wid = lax.axis_index("s") * NC + lax.axis_index("c")
        base = wid * rows_per_tile
        _zero_i32(hist, L * B1)
        laneb = lax.iota(jnp.int32, L) * B1
        one16 = jnp.ones((L,), jnp.int32)
        sx = (sx0, sx1)
        sy = (sy0, sy1)
        for b in range(2):
            _start_pair(xh, yh, xb, yb, sx[b], sy[b], b, base + b * CROWS)

        @pl.loop(0, nchunk, step=2)
        def _chunks(g):
            for b in range(2):
                c = g + b
                _wait_pair(xh, yh, xb, yb, sx[b], sy[b], b)
                for r in range(CROWS):

                    @plsc.parallel_loop(0, csteps, unroll=UNROLL)
                    def _steps(i):
                        s = pl.ds(i * L, L)
                        d = xb[b, r, s] - yb[b, r, s]
                        v = d * d
                        bits = lax.bitcast_convert_type(v, jnp.int32)
                        hi = lax.shift_right_logical(bits, 20)
                        plsc.addupdate_scatter(hist, [laneb + hi], one16)

                @pl.when(c + 2 < nchunk)
                def _():
                    _start_pair(
                        xh, yh, xb, yb, sx[b], sy[b], b,
                        base + (c + 2) * CROWS,
                    )

        pltpu.sync_copy(hist, out_h.at[wid])

    return body


def _sc_pass2(nrows):
    rows_per_tile = nrows // NW
    nchunk = rows_per_tile // CROWS
    csteps = COLS // L

    @functools.partial(
        pl.kernel,
        out_type=(
            jax.ShapeDtypeStruct((NW, L * B2), jnp.int32),
            jax.ShapeDtypeStruct((NW, L), jnp.float32),
            jax.ShapeDtypeStruct((NW, L), jnp.float32),
        ),
        mesh=_mesh(),
        compiler_params=pltpu.CompilerParams(needs_layout_passes=False),
        scratch_types=[
            pltpu.VMEM((2, CROWS, COLS), jnp.float32),
            pltpu.VMEM((2, CROWS, COLS), jnp.float32),
            pltpu.VMEM((L * B2,), jnp.int32),
            pltpu.VMEM((L,), jnp.int32),
            pltpu.VMEM((L,), jnp.float32),
            pltpu.VMEM((L,), jnp.float32),
            pltpu.SemaphoreType.DMA,
            pltpu.SemaphoreType.DMA,
            pltpu.SemaphoreType.DMA,
            pltpu.SemaphoreType.DMA,
        ],
    )
    def body(xh, yh, b1h, out_h, outl_h, outy_h, xb, yb, hist, b1s, accl, accy,
             sx0, sx1, sy0, sy1):
        wid = lax.axis_index("s") * NC + lax.axis_index("c")
        base = wid * rows_per_tile
        _zero_i32(hist, L * B2)
        pltpu.sync_copy(b1h, b1s)
        b1v = b1s[...]
        laneb = lax.iota(jnp.int32, L) * B2
        one16 = jnp.ones((L,), jnp.int32)
        zf = jnp.zeros((L,), jnp.float32)
        accl[...] = zf
        accy[...] = zf
        sx = (sx0, sx1)
        sy = (sy0, sy1)
        for b in range(2):
            _start_pair(xh, yh, xb, yb, sx[b], sy[b], b, base + b * CROWS)

        @pl.loop(0, nchunk, step=2)
        def _chunks(g):
            for b in range(2):
                c = g + b
                _wait_pair(xh, yh, xb, yb, sx[b], sy[b], b)
                carry = (jnp.zeros((L,), jnp.float32),
                         jnp.zeros((L,), jnp.float32))
                for r in range(CROWS):

                    @plsc.parallel_loop(0, csteps, unroll=UNROLL, carry=carry)
                    def _steps(i, cr):
                        al, ay = cr
                        s = pl.ds(i * L, L)
                        yv = yb[b, r, s]
                        d = xb[b, r, s] - yv
                        v = d * d
                        y2 = yv * yv
                        bits = lax.bitcast_convert_type(v, jnp.int32)
                        hi12 = lax.shift_right_logical(bits, 20)
                        m_gt = hi12 > b1v
                        al = al + jnp.where(m_gt, v, 0.0)
                        ay = ay + jnp.where(m_gt, y2, 0.0)
                        m_eq = hi12 == b1v
                        mid = lax.shift_right_logical(bits, 8) & 0xFFF
                        plsc.addupdate_scatter(
                            hist, [laneb + mid], one16, mask=m_eq
                        )
                        return (al, ay)

                    carry = _steps
                accl[...] = accl[...] + carry[0]
                accy[...] = accy[...] + carry[1]

                @pl.when(c + 2 < nchunk)
                def _():
                    _start_pair(
                        xh, yh, xb, yb, sx[b], sy[b], b,
                        base + (c + 2) * CROWS,
                    )

        pltpu.sync_copy(hist, out_h.at[wid])
        pltpu.sync_copy(accl, outl_h.at[wid])
        pltpu.sync_copy(accy, outy_h.at[wid])

    return body


def _sc_pass3(nrows):
    rows_per_tile = nrows // NW
    nchunk = rows_per_tile // CROWS
    csteps = COLS // L

    @functools.partial(
        pl.kernel,
        out_type=(
            jax.ShapeDtypeStruct((NW, L * B3), jnp.int32),
            jax.ShapeDtypeStruct((NW, L * B3), jnp.float32),
            jax.ShapeDtypeStruct((NW, L), jnp.float32),
            jax.ShapeDtypeStruct((NW, L), jnp.float32),
        ),
        mesh=_mesh(),
        compiler_params=pltpu.CompilerParams(needs_layout_passes=False),
        scratch_types=[
            pltpu.VMEM((2, CROWS, COLS), jnp.float32),
            pltpu.VMEM((2, CROWS, COLS), jnp.float32),
            pltpu.VMEM((L * B3,), jnp.int32),
            pltpu.VMEM((L * B3,), jnp.float32),
            pltpu.VMEM((L,), jnp.int32),
            pltpu.VMEM((L,), jnp.float32),
            pltpu.VMEM((L,), jnp.float32),
            pltpu.SemaphoreType.DMA,
            pltpu.SemaphoreType.DMA,
            pltpu.SemaphoreType.DMA,
            pltpu.SemaphoreType.DMA,
        ],
    )
    def body(xh, yh, t24h, outc_h, outy3_h, outl_h, outy_h, xb, yb, hc, hy,
             t24s, accl, accy, sx0, sx1, sy0, sy1):
        wid = lax.axis_index("s") * NC + lax.axis_index("c")
        base = wid * rows_per_tile
        _zero_i32(hc, L * B3)
        _zero_f32(hy, L * B3)
        pltpu.sync_copy(t24h, t24s)
        t24v = t24s[...]
        laneb = lax.iota(jnp.int32, L) * B3
        one16 = jnp.ones((L,), jnp.int32)
        zf = jnp.zeros((L,), jnp.float32)
        accl[...] = zf
        accy[...] = zf
        sx = (sx0, sx1)
        sy = (sy0, sy1)
        for b in range(2):
            _start_pair(xh, yh, xb, yb, sx[b], sy[b], b, base + b * CROWS)

        @pl.loop(0, nchunk, step=2)
        def _chunks(g):
            for b in range(2):
                c = g + b
                _wait_pair(xh, yh, xb, yb, sx[b], sy[b], b)
                carry = (jnp.zeros((L,), jnp.float32),
                         jnp.zeros((L,), jnp.float32))
                for r in range(CROWS):

                    @plsc.parallel_loop(0, csteps, unroll=UNROLL, carry=carry)
                    def _steps(i, cr):
                        al, ay = cr
                        s = pl.ds(i * L, L)
                        yv = yb[b, r, s]
                        d = xb[b, r, s] - yv
                        v = d * d
                        y2 = yv * yv
                        bits = lax.bitcast_convert_type(v, jnp.int32)
                        hi24 = lax.shift_right_logical(bits, 8)
                        du = lax.bitcast_convert_type(hi24 - t24v, jnp.uint32)
                        m_sel = du <= jnp.uint32(0xFFF)
                        al = al + jnp.where(m_sel, v, 0.0)
                        ay = ay + jnp.where(m_sel, y2, 0.0)
                        m_eq = du == jnp.uint32(0)
                        lo8 = bits & 0xFF
                        addr = laneb + lo8
                        plsc.addupdate_scatter(hc, [addr], one16, mask=m_eq)
                        plsc.addupdate_scatter(hy, [addr], y2, mask=m_eq)
                        return (al, ay)

                    carry = _steps
                accl[...] = accl[...] + carry[0]
                accy[...] = accy[...] + carry[1]

                @pl.when(c + 2 < nchunk)
                def _():
                    _start_pair(
                        xh, yh, xb, yb, sx[b], sy[b], b,
                        base + (c + 2) * CROWS,
                    )

        pltpu.sync_copy(hc, outc_h.at[wid])
        pltpu.sync_copy(hy, outy3_h.at[wid])
        pltpu.sync_copy(accl, outl_h.at[wid])
        pltpu.sync_copy(accy, outy_h.at[wid])

    return body


def _suffix_select(counts2d, kf):
    r = counts2d.shape[0]
    u128 = (
        lax.broadcasted_iota(jnp.int32, (128, 128), 0)
        <= lax.broadcasted_iota(jnp.int32, (128, 128), 1)
    ).astype(jnp.float32)
    cw = jnp.dot(counts2d, u128, preferred_element_type=jnp.float32)
    rowtot = cw[:, 127:128]
    lstrict = (
        lax.broadcasted_iota(jnp.int32, (r, r), 1)
        < lax.broadcasted_iota(jnp.int32, (r, r), 0)
    ).astype(jnp.float32)
    rowpref = jnp.dot(lstrict, rowtot, preferred_element_type=jnp.float32)
    cincl = cw + rowpref
    total = jnp.max(cincl)
    s_ge = total - cincl + counts2d
    bidx = (
        lax.broadcasted_iota(jnp.int32, (r, 128), 0) * 128
        + lax.broadcasted_iota(jnp.int32, (r, 128), 1)
    )
    b = jnp.max(jnp.where(s_ge >= kf, bidx, -1))
    c_above = total - jnp.max(jnp.where(bidx == b, cincl, -1.0))
    return b, c_above, total


def _merge1(part, k):
    def body(part_ref, out_ref):
        p = part_ref[...].astype(jnp.float32)
        counts2d = jnp.sum(p, axis=0)
        b1, c1, _ = _suffix_select(counts2d, jnp.float32(k))
        ri = lax.broadcasted_iota(jnp.int32, (8, 128), 0)
        out_ref[...] = jnp.where(
            ri == 0, b1, jnp.where(ri == 1, c1.astype(jnp.int32), 0)
        )

    return pl.pallas_call(
        body,
        out_shape=jax.ShapeDtypeStruct((8, 128), jnp.int32),
    )(part)


def _merge2(part, s1l, s1y, msg1, k):
    def body(part_ref, s1l_ref, s1y_ref, msg1_ref, out_ref, sums_ref):
        p = part_ref[...].astype(jnp.float32)
        counts2d = jnp.sum(p, axis=0)
        b1 = msg1_ref[0, 0]
        c1 = msg1_ref[1, 0]
        k2f = jnp.float32(k) - c1.astype(jnp.float32)
        b2, c2, _ = _suffix_select(counts2d, k2f)
        t24 = b1 * 4096 + b2
        c12 = c1 + c2.astype(jnp.int32)
        ri = lax.broadcasted_iota(jnp.int32, (8, 128), 0)
        out_ref[...] = jnp.where(ri == 0, t24, jnp.where(ri == 1, c12, 0))
        sl = jnp.sum(s1l_ref[...])
        sy = jnp.sum(s1y_ref[...])
        rf = lax.broadcasted_iota(jnp.int32, (8, 128), 0)
        sums_ref[...] = jnp.where(rf == 0, sl, jnp.where(rf == 1, sy, 0.0))

    return pl.pallas_call(
        body,
        out_shape=(
            jax.ShapeDtypeStruct((8, 128), jnp.int32),
            jax.ShapeDtypeStruct((8, 128), jnp.float32),
        ),
        in_specs=[
            pl.BlockSpec(memory_space=pltpu.VMEM),
            pl.BlockSpec(memory_space=pltpu.VMEM),
            pl.BlockSpec(memory_space=pltpu.VMEM),
            pl.BlockSpec(memory_space=pltpu.SMEM),
        ],
    )(part, s1l, s1y, msg1)


def _merge3(partc, party, s2l, s2y, msg2, sums2, k):
    def body(partc_ref, party_ref, s2l_ref, s2y_ref, msg2_ref, sums2_ref,
             out_ref):
        pc = partc_ref[...].astype(jnp.float32)
        counts2d = jnp.sum(pc, axis=0)
        y2b = jnp.sum(party_ref[...], axis=0)
        t24 = msg2_ref[0, 0]
        c12 = msg2_ref[1, 0]
        k3f = jnp.float32(k) - c12.astype(jnp.float32)
        b3, c3, _ = _suffix_select(counts2d, k3f)
        jf = k3f - c3
        r = counts2d.shape[0]
        bidx = (
            lax.broadcasted_iota(jnp.int32, (r, 128), 0) * 128
            + lax.broadcasted_iota(jnp.int32, (r, 128), 1)
        )
        vals = lax.bitcast_convert_type(t24 * 256 + bidx, jnp.float32)
        nz = counts2d > 0.0
        above = (bidx > b3) & nz
        s3l_above = jnp.sum(jnp.where(above, vals * counts2d, 0.0))
        s3y_above = jnp.sum(jnp.where(above, y2b, 0.0))
        s3l_all = jnp.sum(jnp.where(nz, vals * counts2d, 0.0))
        s3y_all = jnp.sum(y2b)
        at_b3 = bidx == b3
        cb3 = jnp.max(jnp.where(at_b3, counts2d, -1.0))
        yb3 = jnp.max(jnp.where(at_b3, y2b, -1.0))
        vb3 = jnp.max(jnp.where(at_b3, vals, -1.0))
        s2l_tot = jnp.sum(s2l_ref[...])
        s2y_tot = jnp.sum(s2y_ref[...])
        s1l = sums2_ref[0, 0]
        s1y = sums2_ref[1, 0]
        num = s1l + (s2l_tot - s3l_all) + s3l_above + jf * vb3
        den = s1y + (s2y_tot - s3y_all) + s3y_above + jf * yb3 / cb3
        res = 0.5 * num / den
        ri = lax.broadcasted_iota(jnp.int32, (8, 128), 0)
        out_ref[...] = jnp.where(ri == 0, res, 0.0)

    return pl.pallas_call(
        body,
        out_shape=jax.ShapeDtypeStruct((8, 128), jnp.float32),
        in_specs=[
            pl.BlockSpec(memory_space=pltpu.VMEM),
            pl.BlockSpec(memory_space=pltpu.VMEM),
            pl.BlockSpec(memory_space=pltpu.VMEM),
            pl.BlockSpec(memory_space=pltpu.VMEM),
            pl.BlockSpec(memory_space=pltpu.SMEM),
            pl.BlockSpec(memory_space=pltpu.SMEM),
        ],
    )(partc, party, s2l, s2y, msg2, sums2)


def kernel(x, y):
    n = x.size
    k = int(n * 0.1)
    nrows = n // COLS
    xf = x.reshape(nrows, COLS)
    yf = y.reshape(nrows, COLS)

    part1 = _sc_pass1(nrows)(xf, yf)
    msg1 = _merge1(part1.reshape(NW * L, B1 // 128, 128), k)

    b1v = msg1[0, :L]
    part2, s1l, s1y = _sc_pass2(nrows)(xf, yf, b1v)
    msg2, sums2 = _merge2(
        part2.reshape(NW * L, B2 // 128, 128), s1l, s1y, msg1, k
    )

    t24v = msg2[0, :L]
    part3c, part3y, s2l, s2y = _sc_pass3(nrows)(xf, yf, t24v)
    out = _merge3(
        part3c.reshape(NW * L, B3 // 128, 128),
        part3y.reshape(NW * L, B3 // 128, 128),
        s2l, s2y, msg2, sums2, k,
    )
    return out[0, 0]


# speedup vs baseline: 1.5195x; 1.5195x over previous
import functools

import jax
import jax.numpy as jnp
from jax import lax
from jax.experimental import pallas as pl
from jax.experimental.pallas import tpu as pltpu
from jax.experimental.pallas import tpu_sc as plsc

NC = 2
NS = 16
NW = NC * NS
L = 16

COLS = 1024
CROWS = 8
B1 = 4096
B2 = 4096
B3 = 256
UNROLL = 8


def _start_pair(xh, yh, xb, yb, sx, sy, b, row0):
    pltpu.async_copy(xh.at[pl.ds(row0, CROWS)], xb.at[b], sx)
    pltpu.async_copy(yh.at[pl.ds(row0, CROWS)], yb.at[b], sy)


def _wait_pair(xh, yh, xb, yb, sx, sy, b):
    pltpu.make_async_copy(xh.at[pl.ds(0, CROWS)], xb.at[b], sx).wait()
    pltpu.make_async_copy(yh.at[pl.ds(0, CROWS)], yb.at[b], sy).wait()


def _zero_i32(ref, n):
    z = jnp.zeros((L,), jnp.int32)

    @pl.loop(0, n // L, unroll=16)
    def _(i):
        ref[pl.ds(i * L, L)] = z


def _zero_f32(ref, n):
    z = jnp.zeros((L,), jnp.float32)

    @pl.loop(0, n // L, unroll=16)
    def _(i):
        ref[pl.ds(i * L, L)] = z


def _mesh():
    return plsc.VectorSubcoreMesh(
        core_axis_name="c", subcore_axis_name="s", num_cores=NC, num_subcores=NS
    )


def _sc_pass1(nrows):
    rows_per_tile = nrows // NW
    nchunk = rows_per_tile // CROWS
    csteps = COLS // L

    @functools.partial(
        pl.kernel,
        out_type=jax.ShapeDtypeStruct((NW, L * B1), jnp.int32),
        mesh=_mesh(),
        compiler_params=pltpu.CompilerParams(needs_layout_passes=False),
        scratch_types=[
            pltpu.VMEM((2, CROWS, COLS), jnp.float32),
            pltpu.VMEM((2, CROWS, COLS), jnp.float32),
            pltpu.VMEM((L * B1,), jnp.int32),
            pltpu.SemaphoreType.DMA,
            pltpu.SemaphoreType.DMA,
            pltpu.SemaphoreType.DMA,
            pltpu.SemaphoreType.DMA,
        ],
    )
    def body(xh, yh, out_h, xb, yb, hist, sx0, sx1, sy0, sy1):
        wid = lax.axis_index("s") * NC + lax.axis_index("c")
        base = wid * rows_per_tile
        _zero_i32(hist, L * B1)
        laneb = lax.iota(jnp.int32, L) * B1
        one16 = jnp.ones((L,), jnp.int32)
        sx = (sx0, sx1)
        sy = (sy0, sy1)
        for b in range(2):
            _start_pair(xh, yh, xb, yb, sx[b], sy[b], b, base + b * CROWS)

        @pl.loop(0, nchunk, step=2)
        def _chunks(g):
            for b in range(2):
                c = g + b
                _wait_pair(xh, yh, xb, yb, sx[b], sy[b], b)
                if True:

                    @plsc.parallel_loop(0, csteps, unroll=2)
                    def _steps(i):
                        s = pl.ds(i * L, L)
                        for r in range(CROWS):
                            d = xb[b, r, s] - yb[b, r, s]
                            v = d * d
                            bits = lax.bitcast_convert_type(v, jnp.int32)
                            hi = lax.shift_right_logical(bits, 20)
                            plsc.addupdate_scatter(hist, [laneb + hi], one16)

                @pl.when(c + 2 < nchunk)
                def _():
                    _start_pair(
                        xh, yh, xb, yb, sx[b], sy[b], b,
                        base + (c + 2) * CROWS,
                    )

        pltpu.sync_copy(hist, out_h.at[wid])

    return body


def _sc_pass2(nrows):
    rows_per_tile = nrows // NW
    nchunk = rows_per_tile // CROWS
    csteps = COLS // L

    @functools.partial(
        pl.kernel,
        out_type=(
            jax.ShapeDtypeStruct((NW, L * B2), jnp.int32),
            jax.ShapeDtypeStruct((NW, L), jnp.float32),
            jax.ShapeDtypeStruct((NW, L), jnp.float32),
        ),
        mesh=_mesh(),
        compiler_params=pltpu.CompilerParams(needs_layout_passes=False),
        scratch_types=[
            pltpu.VMEM((2, CROWS, COLS), jnp.float32),
            pltpu.VMEM((2, CROWS, COLS), jnp.float32),
            pltpu.VMEM((L * B2,), jnp.int32),
            pltpu.VMEM((L,), jnp.int32),
            pltpu.VMEM((L,), jnp.float32),
            pltpu.VMEM((L,), jnp.float32),
            pltpu.SemaphoreType.DMA,
            pltpu.SemaphoreType.DMA,
            pltpu.SemaphoreType.DMA,
            pltpu.SemaphoreType.DMA,
        ],
    )
    def body(xh, yh, b1h, out_h, outl_h, outy_h, xb, yb, hist, b1s, accl, accy,
             sx0, sx1, sy0, sy1):
        wid = lax.axis_index("s") * NC + lax.axis_index("c")
        base = wid * rows_per_tile
        _zero_i32(hist, L * B2)
        pltpu.sync_copy(b1h, b1s)
        b1v = b1s[...]
        laneb = lax.iota(jnp.int32, L) * B2
        one16 = jnp.ones((L,), jnp.int32)
        zf = jnp.zeros((L,), jnp.float32)
        accl[...] = zf
        accy[...] = zf
        sx = (sx0, sx1)
        sy = (sy0, sy1)
        for b in range(2):
            _start_pair(xh, yh, xb, yb, sx[b], sy[b], b, base + b * CROWS)

        @pl.loop(0, nchunk, step=2)
        def _chunks(g):
            for b in range(2):
                c = g + b
                _wait_pair(xh, yh, xb, yb, sx[b], sy[b], b)
                carry = (jnp.zeros((L,), jnp.float32),
                         jnp.zeros((L,), jnp.float32))
                if True:

                    @plsc.parallel_loop(0, csteps, unroll=2, carry=carry)
                    def _steps(i, cr):
                        al, ay = cr
                        s = pl.ds(i * L, L)
                        for r in range(CROWS):
                            yv = yb[b, r, s]
                            d = xb[b, r, s] - yv
                            v = d * d
                            y2 = yv * yv
                            bits = lax.bitcast_convert_type(v, jnp.int32)
                            hi12 = lax.shift_right_logical(bits, 20)
                            m_gt = hi12 > b1v
                            al = al + jnp.where(m_gt, v, 0.0)
                            ay = ay + jnp.where(m_gt, y2, 0.0)
                            m_eq = hi12 == b1v
                            mid = lax.shift_right_logical(bits, 8) & 0xFFF
                            plsc.addupdate_scatter(
                                hist, [laneb + mid], one16, mask=m_eq
                            )
                        return (al, ay)

                    carry = _steps
                accl[...] = accl[...] + carry[0]
                accy[...] = accy[...] + carry[1]

                @pl.when(c + 2 < nchunk)
                def _():
                    _start_pair(
                        xh, yh, xb, yb, sx[b], sy[b], b,
                        base + (c + 2) * CROWS,
                    )

        pltpu.sync_copy(hist, out_h.at[wid])
        pltpu.sync_copy(accl, outl_h.at[wid])
        pltpu.sync_copy(accy, outy_h.at[wid])

    return body


def _sc_pass3(nrows):
    rows_per_tile = nrows // NW
    nchunk = rows_per_tile // CROWS
    csteps = COLS // L

    @functools.partial(
        pl.kernel,
        out_type=(
            jax.ShapeDtypeStruct((NW, L * B3), jnp.int32),
            jax.ShapeDtypeStruct((NW, L * B3), jnp.float32),
            jax.ShapeDtypeStruct((NW, L), jnp.float32),
            jax.ShapeDtypeStruct((NW, L), jnp.float32),
        ),
        mesh=_mesh(),
        compiler_params=pltpu.CompilerParams(needs_layout_passes=False),
        scratch_types=[
            pltpu.VMEM((2, CROWS, COLS), jnp.float32),
            pltpu.VMEM((2, CROWS, COLS), jnp.float32),
            pltpu.VMEM((L * B3,), jnp.int32),
            pltpu.VMEM((L * B3,), jnp.float32),
            pltpu.VMEM((L,), jnp.int32),
            pltpu.VMEM((L,), jnp.float32),
            pltpu.VMEM((L,), jnp.float32),
            pltpu.SemaphoreType.DMA,
            pltpu.SemaphoreType.DMA,
            pltpu.SemaphoreType.DMA,
            pltpu.SemaphoreType.DMA,
        ],
    )
    def body(xh, yh, t24h, outc_h, outy3_h, outl_h, outy_h, xb, yb, hc, hy,
             t24s, accl, accy, sx0, sx1, sy0, sy1):
        wid = lax.axis_index("s") * NC + lax.axis_index("c")
        base = wid * rows_per_tile
        _zero_i32(hc, L * B3)
        _zero_f32(hy, L * B3)
        pltpu.sync_copy(t24h, t24s)
        t24v = t24s[...]
        thiv = t24v | 0xFFF
        laneb = lax.iota(jnp.int32, L) * B3
        one16 = jnp.ones((L,), jnp.int32)
        zf = jnp.zeros((L,), jnp.float32)
        accl[...] = zf
        accy[...] = zf
        sx = (sx0, sx1)
        sy = (sy0, sy1)
        for b in range(2):
            _start_pair(xh, yh, xb, yb, sx[b], sy[b], b, base + b * CROWS)

        @pl.loop(0, nchunk, step=2)
        def _chunks(g):
            for b in range(2):
                c = g + b
                _wait_pair(xh, yh, xb, yb, sx[b], sy[b], b)
                carry = (jnp.zeros((L,), jnp.float32),
                         jnp.zeros((L,), jnp.float32))
                if True:

                    @plsc.parallel_loop(0, csteps, unroll=2, carry=carry)
                    def _steps(i, cr):
                        al, ay = cr
                        s = pl.ds(i * L, L)
                        for r in range(CROWS):
                            yv = yb[b, r, s]
                            d = xb[b, r, s] - yv
                            v = d * d
                            y2 = yv * yv
                            bits = lax.bitcast_convert_type(v, jnp.int32)
                            hi24 = lax.shift_right_logical(bits, 8)
                            m_sel = (hi24 >= t24v) & (hi24 <= thiv)
                            al = al + jnp.where(m_sel, v, 0.0)
                            ay = ay + jnp.where(m_sel, y2, 0.0)
                            m_eq = hi24 == t24v
                            lo8 = bits & 0xFF
                            addr = laneb + lo8
                            plsc.addupdate_scatter(hc, [addr], one16, mask=m_eq)
                            plsc.addupdate_scatter(hy, [addr], y2, mask=m_eq)
                        return (al, ay)

                    carry = _steps
                accl[...] = accl[...] + carry[0]
                accy[...] = accy[...] + carry[1]

                @pl.when(c + 2 < nchunk)
                def _():
                    _start_pair(
                        xh, yh, xb, yb, sx[b], sy[b], b,
                        base + (c + 2) * CROWS,
                    )

        pltpu.sync_copy(hc, outc_h.at[wid])
        pltpu.sync_copy(hy, outy3_h.at[wid])
        pltpu.sync_copy(accl, outl_h.at[wid])
        pltpu.sync_copy(accy, outy_h.at[wid])

    return body


def _suffix_select(counts2d, kf):
    r = counts2d.shape[0]
    u128 = (
        lax.broadcasted_iota(jnp.int32, (128, 128), 0)
        <= lax.broadcasted_iota(jnp.int32, (128, 128), 1)
    ).astype(jnp.float32)
    cw = jnp.dot(counts2d, u128, preferred_element_type=jnp.float32)
    rowtot = cw[:, 127:128]
    lstrict = (
        lax.broadcasted_iota(jnp.int32, (r, r), 1)
        < lax.broadcasted_iota(jnp.int32, (r, r), 0)
    ).astype(jnp.float32)
    rowpref = jnp.dot(lstrict, rowtot, preferred_element_type=jnp.float32)
    cincl = cw + rowpref
    total = jnp.max(cincl)
    s_ge = total - cincl + counts2d
    bidx = (
        lax.broadcasted_iota(jnp.int32, (r, 128), 0) * 128
        + lax.broadcasted_iota(jnp.int32, (r, 128), 1)
    )
    b = jnp.max(jnp.where(s_ge >= kf, bidx, -1))
    c_above = total - jnp.max(jnp.where(bidx == b, cincl, -1.0))
    return b, c_above, total


def _merge1(part, k):
    def body(part_ref, out_ref):
        p = part_ref[...].astype(jnp.float32)
        counts2d = jnp.sum(p, axis=0)
        b1, c1, _ = _suffix_select(counts2d, jnp.float32(k))
        ri = lax.broadcasted_iota(jnp.int32, (8, 128), 0)
        out_ref[...] = jnp.where(
            ri == 0, b1, jnp.where(ri == 1, c1.astype(jnp.int32), 0)
        )

    return pl.pallas_call(
        body,
        out_shape=jax.ShapeDtypeStruct((8, 128), jnp.int32),
    )(part)


def _merge2(part, s1l, s1y, msg1, k):
    def body(part_ref, s1l_ref, s1y_ref, msg1_ref, out_ref, sums_ref):
        p = part_ref[...].astype(jnp.float32)
        counts2d = jnp.sum(p, axis=0)
        b1 = msg1_ref[0, 0]
        c1 = msg1_ref[1, 0]
        k2f = jnp.float32(k) - c1.astype(jnp.float32)
        b2, c2, _ = _suffix_select(counts2d, k2f)
        t24 = b1 * 4096 + b2
        c12 = c1 + c2.astype(jnp.int32)
        ri = lax.broadcasted_iota(jnp.int32, (8, 128), 0)
        out_ref[...] = jnp.where(ri == 0, t24, jnp.where(ri == 1, c12, 0))
        sl = jnp.sum(s1l_ref[...])
        sy = jnp.sum(s1y_ref[...])
        rf = lax.broadcasted_iota(jnp.int32, (8, 128), 0)
        sums_ref[...] = jnp.where(rf == 0, sl, jnp.where(rf == 1, sy, 0.0))

    return pl.pallas_call(
        body,
        out_shape=(
            jax.ShapeDtypeStruct((8, 128), jnp.int32),
            jax.ShapeDtypeStruct((8, 128), jnp.float32),
        ),
        in_specs=[
            pl.BlockSpec(memory_space=pltpu.VMEM),
            pl.BlockSpec(memory_space=pltpu.VMEM),
            pl.BlockSpec(memory_space=pltpu.VMEM),
            pl.BlockSpec(memory_space=pltpu.SMEM),
        ],
    )(part, s1l, s1y, msg1)


def _merge3(partc, party, s2l, s2y, msg2, sums2, k):
    def body(partc_ref, party_ref, s2l_ref, s2y_ref, msg2_ref, sums2_ref,
             out_ref):
        pc = partc_ref[...].astype(jnp.float32)
        counts2d = jnp.sum(pc, axis=0)
        y2b = jnp.sum(party_ref[...], axis=0)
        t24 = msg2_ref[0, 0]
        c12 = msg2_ref[1, 0]
        k3f = jnp.float32(k) - c12.astype(jnp.float32)
        b3, c3, _ = _suffix_select(counts2d, k3f)
        jf = k3f - c3
        r = counts2d.shape[0]
        bidx = (
            lax.broadcasted_iota(jnp.int32, (r, 128), 0) * 128
            + lax.broadcasted_iota(jnp.int32, (r, 128), 1)
        )
        vals = lax.bitcast_convert_type(t24 * 256 + bidx, jnp.float32)
        nz = counts2d > 0.0
        above = (bidx > b3) & nz
        s3l_above = jnp.sum(jnp.where(above, vals * counts2d, 0.0))
        s3y_above = jnp.sum(jnp.where(above, y2b, 0.0))
        s3l_all = jnp.sum(jnp.where(nz, vals * counts2d, 0.0))
        s3y_all = jnp.sum(y2b)
        at_b3 = bidx == b3
        cb3 = jnp.max(jnp.where(at_b3, counts2d, -1.0))
        yb3 = jnp.max(jnp.where(at_b3, y2b, -1.0))
        vb3 = jnp.max(jnp.where(at_b3, vals, -1.0))
        s2l_tot = jnp.sum(s2l_ref[...])
        s2y_tot = jnp.sum(s2y_ref[...])
        s1l = sums2_ref[0, 0]
        s1y = sums2_ref[1, 0]
        num = s1l + (s2l_tot - s3l_all) + s3l_above + jf * vb3
        den = s1y + (s2y_tot - s3y_all) + s3y_above + jf * yb3 / cb3
        res = 0.5 * num / den
        ri = lax.broadcasted_iota(jnp.int32, (8, 128), 0)
        out_ref[...] = jnp.where(ri == 0, res, 0.0)

    return pl.pallas_call(
        body,
        out_shape=jax.ShapeDtypeStruct((8, 128), jnp.float32),
        in_specs=[
            pl.BlockSpec(memory_space=pltpu.VMEM),
            pl.BlockSpec(memory_space=pltpu.VMEM),
            pl.BlockSpec(memory_space=pltpu.VMEM),
            pl.BlockSpec(memory_space=pltpu.VMEM),
            pl.BlockSpec(memory_space=pltpu.SMEM),
            pl.BlockSpec(memory_space=pltpu.SMEM),
        ],
    )(partc, party, s2l, s2y, msg2, sums2)


def kernel(x, y):
    n = x.size
    k = int(n * 0.1)
    nrows = n // COLS
    xf = x.reshape(nrows, COLS)
    yf = y.reshape(nrows, COLS)

    part1 = _sc_pass1(nrows)(xf, yf)
    msg1 = _merge1(part1.reshape(NW * L, B1 // 128, 128), k)

    b1v = msg1[0, :L]
    part2, s1l, s1y = _sc_pass2(nrows)(xf, yf, b1v)
    msg2, sums2 = _merge2(
        part2.reshape(NW * L, B2 // 128, 128), s1l, s1y, msg1, k
    )

    t24v = msg2[0, :L]
    part3c, part3y, s2l, s2y = _sc_pass3(nrows)(xf, yf, t24v)
    out = _merge3(
        part3c.reshape(NW * L, B3 // 128, 128),
        part3y.reshape(NW * L, B3 // 128, 128),
        s2l, s2y, msg2, sums2, k,
    )
    return out[0, 0]


# speedup vs baseline: 1.6008x; 1.0535x over previous
import functools

import jax
import jax.numpy as jnp
from jax import lax
from jax.experimental import pallas as pl
from jax.experimental.pallas import tpu as pltpu
from jax.experimental.pallas import tpu_sc as plsc

NC = 2
NS = 16
NW = NC * NS
L = 16

COLS = 1024
CROWS = 8
B1 = 4096
B2 = 4096
B3 = 256
UNROLL = 4


def _start_pair(xh, yh, xb, yb, sx, sy, b, row0):
    pltpu.async_copy(xh.at[pl.ds(row0, CROWS)], xb.at[b], sx)
    pltpu.async_copy(yh.at[pl.ds(row0, CROWS)], yb.at[b], sy)


def _wait_pair(xh, yh, xb, yb, sx, sy, b):
    pltpu.make_async_copy(xh.at[pl.ds(0, CROWS)], xb.at[b], sx).wait()
    pltpu.make_async_copy(yh.at[pl.ds(0, CROWS)], yb.at[b], sy).wait()


def _zero_i32(ref, n):
    z = jnp.zeros((L,), jnp.int32)

    @pl.loop(0, n // L, unroll=16)
    def _(i):
        ref[pl.ds(i * L, L)] = z


def _zero_f32(ref, n):
    z = jnp.zeros((L,), jnp.float32)

    @pl.loop(0, n // L, unroll=16)
    def _(i):
        ref[pl.ds(i * L, L)] = z


def _mesh():
    return plsc.VectorSubcoreMesh(
        core_axis_name="c", subcore_axis_name="s", num_cores=NC, num_subcores=NS
    )


def _sc_pass1(nrows):
    rows_per_tile = nrows // NW
    nchunk = rows_per_tile // CROWS
    csteps = COLS // L

    @functools.partial(
        pl.kernel,
        out_type=jax.ShapeDtypeStruct((NW, L * B1), jnp.int32),
        mesh=_mesh(),
        compiler_params=pltpu.CompilerParams(needs_layout_passes=False),
        scratch_types=[
            pltpu.VMEM((2, CROWS, COLS), jnp.float32),
            pltpu.VMEM((2, CROWS, COLS), jnp.float32),
            pltpu.VMEM((L * B1,), jnp.int32),
            pltpu.SemaphoreType.DMA,
            pltpu.SemaphoreType.DMA,
            pltpu.SemaphoreType.DMA,
            pltpu.SemaphoreType.DMA,
        ],
    )
    def body(xh, yh, out_h, xb, yb, hist, sx0, sx1, sy0, sy1):
        wid = lax.axis_index("s") * NC + lax.axis_index("c")
        base = wid * rows_per_tile
        _zero_i32(hist, L * B1)
        laneb = lax.iota(jnp.int32, L) * B1
        one16 = jnp.ones((L,), jnp.int32)
        sx = (sx0, sx1)
        sy = (sy0, sy1)
        for b in range(2):
            _start_pair(xh, yh, xb, yb, sx[b], sy[b], b, base + b * CROWS)

        @pl.loop(0, nchunk, step=2)
        def _chunks(g):
            for b in range(2):
                c = g + b
                _wait_pair(xh, yh, xb, yb, sx[b], sy[b], b)
                for r in range(CROWS):

                    @plsc.parallel_loop(0, csteps, unroll=UNROLL)
                    def _steps(i):
                        s = pl.ds(i * L, L)
                        d = xb[b, r, s] - yb[b, r, s]
                        v = d * d
                        bits = lax.bitcast_convert_type(v, jnp.int32)
                        hi = lax.shift_right_logical(bits, 20)
                        plsc.addupdate_scatter(hist, [laneb + hi], one16)

                @pl.when(c + 2 < nchunk)
                def _():
                    _start_pair(
                        xh, yh, xb, yb, sx[b], sy[b], b,
                        base + (c + 2) * CROWS,
                    )

        pltpu.sync_copy(hist, out_h.at[wid])

    return body


def _sc_pass2(nrows):
    rows_per_tile = nrows // NW
    nchunk = rows_per_tile // CROWS
    csteps = COLS // L

    @functools.partial(
        pl.kernel,
        out_type=(
            jax.ShapeDtypeStruct((NW, L * B2), jnp.int32),
            jax.ShapeDtypeStruct((NW, L), jnp.float32),
            jax.ShapeDtypeStruct((NW, L), jnp.float32),
        ),
        mesh=_mesh(),
        compiler_params=pltpu.CompilerParams(needs_layout_passes=False),
        scratch_types=[
            pltpu.VMEM((2, CROWS, COLS), jnp.float32),
            pltpu.VMEM((2, CROWS, COLS), jnp.float32),
            pltpu.VMEM((L * B2,), jnp.int32),
            pltpu.VMEM((L,), jnp.int32),
            pltpu.VMEM((L,), jnp.float32),
            pltpu.VMEM((L,), jnp.float32),
            pltpu.SemaphoreType.DMA,
            pltpu.SemaphoreType.DMA,
            pltpu.SemaphoreType.DMA,
            pltpu.SemaphoreType.DMA,
        ],
    )
    def body(xh, yh, b1h, out_h, outl_h, outy_h, xb, yb, hist, b1s, accl, accy,
             sx0, sx1, sy0, sy1):
        wid = lax.axis_index("s") * NC + lax.axis_index("c")
        base = wid * rows_per_tile
        _zero_i32(hist, L * B2)
        pltpu.sync_copy(b1h, b1s)
        b1v = b1s[...]
        laneb = lax.iota(jnp.int32, L) * B2
        one16 = jnp.ones((L,), jnp.int32)
        zf = jnp.zeros((L,), jnp.float32)
        accl[...] = zf
        accy[...] = zf
        sx = (sx0, sx1)
        sy = (sy0, sy1)
        for b in range(2):
            _start_pair(xh, yh, xb, yb, sx[b], sy[b], b, base + b * CROWS)

        @pl.loop(0, nchunk, step=2)
        def _chunks(g):
            for b in range(2):
                c = g + b
                _wait_pair(xh, yh, xb, yb, sx[b], sy[b], b)
                carry = (jnp.zeros((L,), jnp.float32),
                         jnp.zeros((L,), jnp.float32))
                for r in range(CROWS):

                    @plsc.parallel_loop(0, csteps, unroll=UNROLL, carry=carry)
                    def _steps(i, cr):
                        al, ay = cr
                        s = pl.ds(i * L, L)
                        yv = yb[b, r, s]
                        d = xb[b, r, s] - yv
                        v = d * d
                        y2 = yv * yv
                        bits = lax.bitcast_convert_type(v, jnp.int32)
                        hi12 = lax.shift_right_logical(bits, 20)
                        m_gt = hi12 > b1v
                        al = al + jnp.where(m_gt, v, 0.0)
                        ay = ay + jnp.where(m_gt, y2, 0.0)
                        m_eq = hi12 == b1v
                        mid = lax.shift_right_logical(bits, 8) & 0xFFF
                        plsc.addupdate_scatter(
                            hist, [laneb + mid], one16, mask=m_eq
                        )
                        return (al, ay)

                    carry = _steps
                accl[...] = accl[...] + carry[0]
                accy[...] = accy[...] + carry[1]

                @pl.when(c + 2 < nchunk)
                def _():
                    _start_pair(
                        xh, yh, xb, yb, sx[b], sy[b], b,
                        base + (c + 2) * CROWS,
                    )

        pltpu.sync_copy(hist, out_h.at[wid])
        pltpu.sync_copy(accl, outl_h.at[wid])
        pltpu.sync_copy(accy, outy_h.at[wid])

    return body


def _sc_pass3(nrows):
    rows_per_tile = nrows // NW
    nchunk = rows_per_tile // CROWS
    csteps = COLS // L

    @functools.partial(
        pl.kernel,
        out_type=(
            jax.ShapeDtypeStruct((NW, L * B3), jnp.int32),
            jax.ShapeDtypeStruct((NW, L * B3), jnp.float32),
            jax.ShapeDtypeStruct((NW, L), jnp.float32),
            jax.ShapeDtypeStruct((NW, L), jnp.float32),
        ),
        mesh=_mesh(),
        compiler_params=pltpu.CompilerParams(needs_layout_passes=False),
        scratch_types=[
            pltpu.VMEM((2, CROWS, COLS), jnp.float32),
            pltpu.VMEM((2, CROWS, COLS), jnp.float32),
            pltpu.VMEM((L * B3,), jnp.int32),
            pltpu.VMEM((L * B3,), jnp.float32),
            pltpu.VMEM((L,), jnp.int32),
            pltpu.VMEM((L,), jnp.float32),
            pltpu.VMEM((L,), jnp.float32),
            pltpu.SemaphoreType.DMA,
            pltpu.SemaphoreType.DMA,
            pltpu.SemaphoreType.DMA,
            pltpu.SemaphoreType.DMA,
        ],
    )
    def body(xh, yh, t24h, outc_h, outy3_h, outl_h, outy_h, xb, yb, hc, hy,
             t24s, accl, accy, sx0, sx1, sy0, sy1):
        wid = lax.axis_index("s") * NC + lax.axis_index("c")
        base = wid * rows_per_tile
        _zero_i32(hc, L * B3)
        _zero_f32(hy, L * B3)
        pltpu.sync_copy(t24h, t24s)
        t24v = t24s[...]
        thiv = t24v | 0xFFF
        laneb = lax.iota(jnp.int32, L) * B3
        one16 = jnp.ones((L,), jnp.int32)
        zf = jnp.zeros((L,), jnp.float32)
        accl[...] = zf
        accy[...] = zf
        sx = (sx0, sx1)
        sy = (sy0, sy1)
        for b in range(2):
            _start_pair(xh, yh, xb, yb, sx[b], sy[b], b, base + b * CROWS)

        @pl.loop(0, nchunk, step=2)
        def _chunks(g):
            for b in range(2):
                c = g + b
                _wait_pair(xh, yh, xb, yb, sx[b], sy[b], b)
                carry = (jnp.zeros((L,), jnp.float32),
                         jnp.zeros((L,), jnp.float32))
                for r in range(CROWS):

                    @plsc.parallel_loop(0, csteps, unroll=UNROLL, carry=carry)
                    def _steps(i, cr):
                        al, ay = cr
                        s = pl.ds(i * L, L)
                        yv = yb[b, r, s]
                        d = xb[b, r, s] - yv
                        v = d * d
                        y2 = yv * yv
                        bits = lax.bitcast_convert_type(v, jnp.int32)
                        hi24 = lax.shift_right_logical(bits, 8)
                        m_sel = (hi24 >= t24v) & (hi24 <= thiv)
                        al = al + jnp.where(m_sel, v, 0.0)
                        ay = ay + jnp.where(m_sel, y2, 0.0)
                        m_eq = hi24 == t24v
                        lo8 = bits & 0xFF
                        addr = laneb + lo8
                        plsc.addupdate_scatter(hc, [addr], one16, mask=m_eq)
                        plsc.addupdate_scatter(hy, [addr], y2, mask=m_eq)
                        return (al, ay)

                    carry = _steps
                accl[...] = accl[...] + carry[0]
                accy[...] = accy[...] + carry[1]

                @pl.when(c + 2 < nchunk)
                def _():
                    _start_pair(
                        xh, yh, xb, yb, sx[b], sy[b], b,
                        base + (c + 2) * CROWS,
                    )

        pltpu.sync_copy(hc, outc_h.at[wid])
        pltpu.sync_copy(hy, outy3_h.at[wid])
        pltpu.sync_copy(accl, outl_h.at[wid])
        pltpu.sync_copy(accy, outy_h.at[wid])

    return body


def _suffix_select(counts2d, kf):
    r = counts2d.shape[0]
    u128 = (
        lax.broadcasted_iota(jnp.int32, (128, 128), 0)
        <= lax.broadcasted_iota(jnp.int32, (128, 128), 1)
    ).astype(jnp.float32)
    cw = jnp.dot(counts2d, u128, preferred_element_type=jnp.float32)
    rowtot = cw[:, 127:128]
    lstrict = (
        lax.broadcasted_iota(jnp.int32, (r, r), 1)
        < lax.broadcasted_iota(jnp.int32, (r, r), 0)
    ).astype(jnp.float32)
    rowpref = jnp.dot(lstrict, rowtot, preferred_element_type=jnp.float32)
    cincl = cw + rowpref
    total = jnp.max(cincl)
    s_ge = total - cincl + counts2d
    bidx = (
        lax.broadcasted_iota(jnp.int32, (r, 128), 0) * 128
        + lax.broadcasted_iota(jnp.int32, (r, 128), 1)
    )
    b = jnp.max(jnp.where(s_ge >= kf, bidx, -1))
    c_above = total - jnp.max(jnp.where(bidx == b, cincl, -1.0))
    return b, c_above, total


def _merge1(part, k):
    def body(part_ref, out_ref):
        p = part_ref[...].astype(jnp.float32)
        counts2d = jnp.sum(p, axis=0)
        b1, c1, _ = _suffix_select(counts2d, jnp.float32(k))
        ri = lax.broadcasted_iota(jnp.int32, (8, 128), 0)
        out_ref[...] = jnp.where(
            ri == 0, b1, jnp.where(ri == 1, c1.astype(jnp.int32), 0)
        )

    return pl.pallas_call(
        body,
        out_shape=jax.ShapeDtypeStruct((8, 128), jnp.int32),
    )(part)


def _merge2(part, s1l, s1y, msg1, k):
    def body(part_ref, s1l_ref, s1y_ref, msg1_ref, out_ref, sums_ref):
        p = part_ref[...].astype(jnp.float32)
        counts2d = jnp.sum(p, axis=0)
        b1 = msg1_ref[0, 0]
        c1 = msg1_ref[1, 0]
        k2f = jnp.float32(k) - c1.astype(jnp.float32)
        b2, c2, _ = _suffix_select(counts2d, k2f)
        t24 = b1 * 4096 + b2
        c12 = c1 + c2.astype(jnp.int32)
        ri = lax.broadcasted_iota(jnp.int32, (8, 128), 0)
        out_ref[...] = jnp.where(ri == 0, t24, jnp.where(ri == 1, c12, 0))
        sl = jnp.sum(s1l_ref[...])
        sy = jnp.sum(s1y_ref[...])
        rf = lax.broadcasted_iota(jnp.int32, (8, 128), 0)
        sums_ref[...] = jnp.where(rf == 0, sl, jnp.where(rf == 1, sy, 0.0))

    return pl.pallas_call(
        body,
        out_shape=(
            jax.ShapeDtypeStruct((8, 128), jnp.int32),
            jax.ShapeDtypeStruct((8, 128), jnp.float32),
        ),
        in_specs=[
            pl.BlockSpec(memory_space=pltpu.VMEM),
            pl.BlockSpec(memory_space=pltpu.VMEM),
            pl.BlockSpec(memory_space=pltpu.VMEM),
            pl.BlockSpec(memory_space=pltpu.SMEM),
        ],
    )(part, s1l, s1y, msg1)


def _merge3(partc, party, s2l, s2y, msg2, sums2, k):
    def body(partc_ref, party_ref, s2l_ref, s2y_ref, msg2_ref, sums2_ref,
             out_ref):
        pc = partc_ref[...].astype(jnp.float32)
        counts2d = jnp.sum(pc, axis=0)
        y2b = jnp.sum(party_ref[...], axis=0)
        t24 = msg2_ref[0, 0]
        c12 = msg2_ref[1, 0]
        k3f = jnp.float32(k) - c12.astype(jnp.float32)
        b3, c3, _ = _suffix_select(counts2d, k3f)
        jf = k3f - c3
        r = counts2d.shape[0]
        bidx = (
            lax.broadcasted_iota(jnp.int32, (r, 128), 0) * 128
            + lax.broadcasted_iota(jnp.int32, (r, 128), 1)
        )
        vals = lax.bitcast_convert_type(t24 * 256 + bidx, jnp.float32)
        nz = counts2d > 0.0
        above = (bidx > b3) & nz
        s3l_above = jnp.sum(jnp.where(above, vals * counts2d, 0.0))
        s3y_above = jnp.sum(jnp.where(above, y2b, 0.0))
        s3l_all = jnp.sum(jnp.where(nz, vals * counts2d, 0.0))
        s3y_all = jnp.sum(y2b)
        at_b3 = bidx == b3
        cb3 = jnp.max(jnp.where(at_b3, counts2d, -1.0))
        yb3 = jnp.max(jnp.where(at_b3, y2b, -1.0))
        vb3 = jnp.max(jnp.where(at_b3, vals, -1.0))
        s2l_tot = jnp.sum(s2l_ref[...])
        s2y_tot = jnp.sum(s2y_ref[...])
        s1l = sums2_ref[0, 0]
        s1y = sums2_ref[1, 0]
        num = s1l + (s2l_tot - s3l_all) + s3l_above + jf * vb3
        den = s1y + (s2y_tot - s3y_all) + s3y_above + jf * yb3 / cb3
        res = 0.5 * num / den
        ri = lax.broadcasted_iota(jnp.int32, (8, 128), 0)
        out_ref[...] = jnp.where(ri == 0, res, 0.0)

    return pl.pallas_call(
        body,
        out_shape=jax.ShapeDtypeStruct((8, 128), jnp.float32),
        in_specs=[
            pl.BlockSpec(memory_space=pltpu.VMEM),
            pl.BlockSpec(memory_space=pltpu.VMEM),
            pl.BlockSpec(memory_space=pltpu.VMEM),
            pl.BlockSpec(memory_space=pltpu.VMEM),
            pl.BlockSpec(memory_space=pltpu.SMEM),
            pl.BlockSpec(memory_space=pltpu.SMEM),
        ],
    )(partc, party, s2l, s2y, msg2, sums2)


def kernel(x, y):
    n = x.size
    k = int(n * 0.1)
    nrows = n // COLS
    xf = x.reshape(nrows, COLS)
    yf = y.reshape(nrows, COLS)

    part1 = _sc_pass1(nrows)(xf, yf)
    msg1 = _merge1(part1.reshape(NW * L, B1 // 128, 128), k)

    b1v = msg1[0, :L]
    part2, s1l, s1y = _sc_pass2(nrows)(xf, yf, b1v)
    msg2, sums2 = _merge2(
        part2.reshape(NW * L, B2 // 128, 128), s1l, s1y, msg1, k
    )

    t24v = msg2[0, :L]
    part3c, part3y, s2l, s2y = _sc_pass3(nrows)(xf, yf, t24v)
    out = _merge3(
        part3c.reshape(NW * L, B3 // 128, 128),
        part3y.reshape(NW * L, B3 // 128, 128),
        s2l, s2y, msg2, sums2, k,
    )
    return out[0, 0]
